# baseline (speedup 1.0000x reference)
import functools

import jax
import jax.numpy as jnp
from jax import lax
from jax.experimental import pallas as pl
from jax.experimental.pallas import tpu as pltpu
from jax.experimental.pallas import tpu_sc as plsc

S, HW, B, C = 50, 64, 64, 64
DEC = 16
P = S * HW
N_GATHER = P * B
OUT_ROWS = (DEC + P) * B
DEC_ROWS = DEC * B

NC, NS = 2, 16
NW = NC * NS
ROWS_PER_W = N_GATHER // NW
CHUNK = 640
NCHUNK = ROWS_PER_W // CHUNK
G_PER_CHUNK = CHUNK // B
IDXW = 128
IDX_PER_CHUNK = CHUNK // IDXW


def _sc_body(W_hbm, dec_hbm, pos_hbm, x_hbm, out_hbm, idx_v, emb_v, pos_v,
             sem):
    wid = lax.axis_index("s") * NC + lax.axis_index("c")

    @pl.when(wid < DEC)
    def _dec():
        pltpu.sync_copy(dec_hbm.at[pl.ds(wid * C, C)], pos_v.at[pl.ds(0, C)])
        dv = [pos_v[pl.ds(16 * j, 16)] for j in range(4)]

        def rep_row(r, _):
            for j in range(4):
                emb_v[r, pl.ds(16 * j, 16)] = dv[j]
            return 0

        lax.fori_loop(0, B, rep_row, 0)
        pltpu.sync_copy(emb_v.at[pl.ds(0, B)], out_hbm.at[pl.ds(wid * B, B)])

    def chunk_body(c, _):
        base = wid * ROWS_PER_W + c * CHUNK
        pltpu.sync_copy(x_hbm.at[pl.ds(base, CHUNK)], idx_v)
        waits = [
            pltpu.async_copy(
                W_hbm.at[idx_v.at[pl.ds(k * IDXW, IDXW)]],
                emb_v.at[pl.ds(k * IDXW, IDXW)],
                sem,
            )
            for k in range(IDX_PER_CHUNK)
        ]
        pltpu.sync_copy(pos_hbm.at[pl.ds(base, CHUNK)], pos_v)
        for w in waits:
            w.wait()
        for g in range(G_PER_CHUNK):
            pv = [pos_v[pl.ds(g * C + 16 * j, 16)] for j in range(4)]

            def add_rows(i, _, g=g, pv=pv):
                for rr in range(8):
                    r = g * B + i * 8 + rr
                    for j in range(4):
                        plsc.addupdate(emb_v.at[r, pl.ds(16 * j, 16)], pv[j])
                return 0

            lax.fori_loop(0, B // 8, add_rows, 0)
        pltpu.sync_copy(emb_v, out_hbm.at[pl.ds(DEC_ROWS + base, CHUNK)])
        return 0

    lax.fori_loop(0, NCHUNK, chunk_body, 0)


@jax.jit
def _run(W_emb, dec1, pos1, x1):
    mesh = plsc.VectorSubcoreMesh(core_axis_name="c", subcore_axis_name="s")
    f = functools.partial(
        pl.kernel,
        out_type=jax.ShapeDtypeStruct((OUT_ROWS, C), jnp.float32),
        mesh=mesh,
        scratch_types=[
            pltpu.VMEM((CHUNK,), jnp.int32),
            pltpu.VMEM((CHUNK, C), jnp.float32),
            pltpu.VMEM((CHUNK,), jnp.float32),
            pltpu.SemaphoreType.DMA,
        ],
        compiler_params=pltpu.CompilerParams(use_tc_tiling_on_sc=False),
    )(_sc_body)
    return f(W_emb, dec1, pos1, x1)


def kernel(W_emb, dec_emb, pos, x):
    out = _run(W_emb, dec_emb.reshape(DEC * C), pos.reshape(P * C),
               x.reshape(N_GATHER))
    return out.reshape(DEC + P, B, C)


# speedup vs baseline: 1.0110x; 1.0110x over previous
import functools

import jax
import jax.numpy as jnp
from jax import lax
from jax.experimental import pallas as pl
from jax.experimental.pallas import tpu as pltpu
from jax.experimental.pallas import tpu_sc as plsc

S, HW, B, C = 50, 64, 64, 64
DEC = 16
P = S * HW
N_GATHER = P * B
OUT_ROWS = (DEC + P) * B
DEC_ROWS = DEC * B

NC, NS = 2, 16
NW = NC * NS
ROWS_PER_W = N_GATHER // NW
CHUNK = 640
NCHUNK = ROWS_PER_W // CHUNK
G_PER_CHUNK = CHUNK // B
IDXW = 128
IDX_PER_CHUNK = CHUNK // IDXW


def _sc_body(W_hbm, dec_hbm, pos_hbm, x_hbm, out_hbm, idx_v, emb_v, pos_v,
             sem, idx2_v, emb2_v, pos2_v, sem2):
    wid = lax.axis_index("s") * NC + lax.axis_index("c")

    @pl.when(wid < DEC)
    def _dec():
        pltpu.sync_copy(dec_hbm.at[pl.ds(wid * C, C)], pos_v.at[pl.ds(0, C)])
        dv = [pos_v[pl.ds(16 * j, 16)] for j in range(4)]

        def rep_row(r, _):
            for j in range(4):
                emb_v[r, pl.ds(16 * j, 16)] = dv[j]
            return 0

        lax.fori_loop(0, B, rep_row, 0)
        pltpu.sync_copy(emb_v.at[pl.ds(0, B)], out_hbm.at[pl.ds(wid * B, B)])

    bufs = [(idx_v, emb_v, pos_v, sem), (idx2_v, emb2_v, pos2_v, sem2)]

    def fire(c, b):
        idx_b, emb_b, pos_b, sem_b = b
        base = wid * ROWS_PER_W + c * CHUNK
        pltpu.sync_copy(x_hbm.at[pl.ds(base, CHUNK)], idx_b)
        ws = [
            pltpu.async_copy(
                W_hbm.at[idx_b.at[pl.ds(k * IDXW, IDXW)]],
                emb_b.at[pl.ds(k * IDXW, IDXW)],
                sem_b,
            )
            for k in range(IDX_PER_CHUNK)
        ]
        pltpu.sync_copy(pos_hbm.at[pl.ds(base, CHUNK)], pos_b)
        return ws

    ws = fire(0, bufs[0])
    for c in range(NCHUNK):
        _, emb_b, pos_b, _ = bufs[c % 2]
        ws_next = fire(c + 1, bufs[(c + 1) % 2]) if c + 1 < NCHUNK else []
        for w in ws:
            w.wait()
        for g in range(G_PER_CHUNK):
            pv = [pos_b[pl.ds(g * C + 16 * j, 16)] for j in range(4)]

            def add_rows(i, _, g=g, pv=pv, emb_b=emb_b):
                for rr in range(8):
                    r = g * B + i * 8 + rr
                    for j in range(4):
                        plsc.addupdate(emb_b.at[r, pl.ds(16 * j, 16)], pv[j])
                return 0

            lax.fori_loop(0, B // 8, add_rows, 0)
        base = wid * ROWS_PER_W + c * CHUNK
        pltpu.sync_copy(emb_b, out_hbm.at[pl.ds(DEC_ROWS + base, CHUNK)])
        ws = ws_next


@jax.jit
def _run(W_emb, dec1, pos1, x1):
    mesh = plsc.VectorSubcoreMesh(core_axis_name="c", subcore_axis_name="s")
    f = functools.partial(
        pl.kernel,
        out_type=jax.ShapeDtypeStruct((OUT_ROWS, C), jnp.float32),
        mesh=mesh,
        scratch_types=[
            pltpu.VMEM((CHUNK,), jnp.int32),
            pltpu.VMEM((CHUNK, C), jnp.float32),
            pltpu.VMEM((CHUNK,), jnp.float32),
            pltpu.SemaphoreType.DMA,
            pltpu.VMEM((CHUNK,), jnp.int32),
            pltpu.VMEM((CHUNK, C), jnp.float32),
            pltpu.VMEM((CHUNK,), jnp.float32),
            pltpu.SemaphoreType.DMA,
        ],
        compiler_params=pltpu.CompilerParams(use_tc_tiling_on_sc=False),
    )(_sc_body)
    return f(W_emb, dec1, pos1, x1)


def kernel(W_emb, dec_emb, pos, x):
    out = _run(W_emb, dec_emb.reshape(DEC * C), pos.reshape(P * C),
               x.reshape(N_GATHER))
    return out.reshape(DEC + P, B, C)
